# Initial kernel scaffold; baseline (speedup 1.0000x reference)
import jax
import jax.numpy as jnp
from jax.experimental import pallas as pl


def kernel(ego_states, x, edge_index, batch, gin0_w1, gin0_b1, gin0_w2, gin0_b2, gin1_w1, gin1_b1, gin1_w2, gin1_b2, gin2_w1, gin2_b1, gin2_w2, gin2_b2, mlp_w1, mlp_b1, mlp_w2, mlp_b2):
    raise NotImplementedError("write your pallas kernel here")


# speedup vs baseline: 2.9872x; 2.9872x over previous
import functools

import jax
import jax.numpy as jnp
from jax import lax
from jax.experimental import pallas as pl
from jax.experimental.pallas import tpu as pltpu
from jax.experimental.pallas import tpu_sc as plsc

N = 10000
D = 128
G = 64
E = 320000

NC = 2
NS = 16
NW = NC * NS
CE = 128
CPW = 80
E_PAD = NW * CPW * CE
RPT = 632
TRASH = N
AGG_ROWS = NS * RPT

def _sc_segsum_body(h_hbm, src_hbm, dst_hbm, zeros_hbm, out_hbm,
                    sidx, didx, rows, agg, sem):
    c = lax.axis_index("c")
    s = lax.axis_index("s")
    w = c * NS + s
    pltpu.sync_copy(zeros_hbm, agg.at[pl.ds(s * RPT, RPT)])
    pltpu.sync_copy(src_hbm.at[pl.ds(w * CPW, CPW)], sidx)
    pltpu.sync_copy(dst_hbm.at[pl.ds(w * CPW, CPW)], didx)
    plsc.subcore_barrier()

    def body(j, carry):
        pltpu.async_copy(h_hbm.at[sidx.at[j]], rows, sem).wait()
        pltpu.sync_copy(rows, agg.at[didx.at[j]], add=True)
        return carry

    lax.fori_loop(0, CPW, body, 0)
    plsc.subcore_barrier()
    pltpu.sync_copy(agg.at[pl.ds(s * RPT, RPT)],
                    out_hbm.at[c, pl.ds(s * RPT, RPT)])


@functools.cache
def _sc_segsum_kernel():
    mesh = plsc.VectorSubcoreMesh(core_axis_name="c", subcore_axis_name="s",
                                  num_cores=NC, num_subcores=NS)
    return pl.kernel(
        _sc_segsum_body,
        out_type=jax.ShapeDtypeStruct((NC, AGG_ROWS, D), jnp.float32),
        mesh=mesh,
        scratch_types=[
            pltpu.VMEM((CPW, CE), jnp.int32),
            pltpu.VMEM((CPW, CE), jnp.int32),
            pltpu.VMEM((CE, D), jnp.float32),
            pltpu.VMEM_SHARED((AGG_ROWS, D), jnp.float32),
            pltpu.SemaphoreType.DMA,
        ],
    )


def _sc_segsum(h, src_p, dst_p, zeros):
    return _sc_segsum_kernel()(h, src_p, dst_p, zeros)


BM = 1000
NB = N // BM


def _mlp_body(relu_out, h_ref, agg_ref, w1_ref, b1_ref, w2_ref, b2_ref, o_ref):
    z = h_ref[...] + agg_ref[0] + agg_ref[1]
    y = jnp.dot(z, w1_ref[...], preferred_element_type=jnp.float32) + b1_ref[...]
    y = jnp.maximum(y, 0.0)
    y = jnp.dot(y, w2_ref[...], preferred_element_type=jnp.float32) + b2_ref[...]
    if relu_out:
        y = jnp.maximum(y, 0.0)
    o_ref[...] = y


def _tc_mlp(h, agg, w1, b1, w2, b2, relu_out):
    return pl.pallas_call(
        functools.partial(_mlp_body, relu_out),
        grid=(NB,),
        in_specs=[
            pl.BlockSpec((BM, D), lambda i: (i, 0)),
            pl.BlockSpec((NC, BM, D), lambda i: (0, i, 0)),
            pl.BlockSpec((D, D), lambda i: (0, 0)),
            pl.BlockSpec((1, D), lambda i: (0, 0)),
            pl.BlockSpec((D, D), lambda i: (0, 0)),
            pl.BlockSpec((1, D), lambda i: (0, 0)),
        ],
        out_specs=pl.BlockSpec((BM, D), lambda i: (i, 0)),
        out_shape=jax.ShapeDtypeStruct((N, D), jnp.float32),
    )(h, agg, w1, b1, w2, b2)


def _final_body(h_ref, agg_ref, w1_ref, b1_ref, w2_ref, b2_ref, batch_ref,
                ego_ref, mw1_ref, mb1_ref, mw2_ref, mb2_ref, o_ref, acc, cnt):
    i = pl.program_id(0)

    @pl.when(i == 0)
    def _():
        acc[...] = jnp.zeros_like(acc)
        cnt[...] = jnp.zeros_like(cnt)

    z = h_ref[...] + agg_ref[0] + agg_ref[1]
    y = jnp.maximum(
        jnp.dot(z, w1_ref[...], preferred_element_type=jnp.float32) + b1_ref[...], 0.0)
    h3 = jnp.dot(y, w2_ref[...], preferred_element_type=jnp.float32) + b2_ref[...]

    bb = batch_ref[0]
    gi = lax.broadcasted_iota(jnp.int32, (G, BM), 0)
    pt = jnp.where(gi == bb, 1.0, 0.0)
    acc[...] += lax.dot_general(pt, h3, (((1,), (0,)), ((), ())),
                                preferred_element_type=jnp.float32)
    cnt[...] += jnp.broadcast_to(jnp.sum(pt, axis=1, keepdims=True), (G, D))

    @pl.when(i == NB - 1)
    def _():
        pooled = acc[...] / jnp.maximum(cnt[...], 1.0)
        e = jnp.maximum(
            jnp.dot(ego_ref[...], mw1_ref[...], preferred_element_type=jnp.float32)
            + mb1_ref[...], 0.0)
        e = jnp.dot(e, mw2_ref[...], preferred_element_type=jnp.float32) + mb2_ref[...]
        o_ref[...] = jnp.concatenate([pooled, e], axis=1)


def _tc_final(h, agg, w1, b1, w2, b2, batch3, ego, mw1, mb1, mw2, mb2):
    return pl.pallas_call(
        _final_body,
        grid=(NB,),
        in_specs=[
            pl.BlockSpec((BM, D), lambda i: (i, 0)),
            pl.BlockSpec((NC, BM, D), lambda i: (0, i, 0)),
            pl.BlockSpec((D, D), lambda i: (0, 0)),
            pl.BlockSpec((1, D), lambda i: (0, 0)),
            pl.BlockSpec((D, D), lambda i: (0, 0)),
            pl.BlockSpec((1, D), lambda i: (0, 0)),
            pl.BlockSpec((1, 1, BM), lambda i: (i, 0, 0)),
            pl.BlockSpec((G, D), lambda i: (0, 0)),
            pl.BlockSpec((D, D), lambda i: (0, 0)),
            pl.BlockSpec((1, D), lambda i: (0, 0)),
            pl.BlockSpec((D, D), lambda i: (0, 0)),
            pl.BlockSpec((1, D), lambda i: (0, 0)),
        ],
        out_specs=pl.BlockSpec((G, 2 * D), lambda i: (0, 0)),
        out_shape=jax.ShapeDtypeStruct((G, 2 * D), jnp.float32),
        scratch_shapes=[
            pltpu.VMEM((G, D), jnp.float32),
            pltpu.VMEM((G, D), jnp.float32),
        ],
    )(h, agg, w1, b1, w2, b2, batch3, ego, mw1, mb1, mw2, mb2)


def kernel(ego_states, x, edge_index, batch,
           gin0_w1, gin0_b1, gin0_w2, gin0_b2,
           gin1_w1, gin1_b1, gin1_w2, gin1_b2,
           gin2_w1, gin2_b1, gin2_w2, gin2_b2,
           mlp_w1, mlp_b1, mlp_w2, mlp_b2):
    pad = E_PAD - E
    src_p = jnp.concatenate(
        [edge_index[0], jnp.zeros((pad,), jnp.int32)]).reshape(NW * CPW, CE)
    dst_p = jnp.concatenate(
        [edge_index[1], jnp.full((pad,), TRASH, jnp.int32)]).reshape(NW * CPW, CE)
    zeros = jnp.zeros((RPT, D), jnp.float32)
    batch3 = batch.reshape(NB, 1, BM)

    b = [v.reshape(1, D) for v in
         (gin0_b1, gin0_b2, gin1_b1, gin1_b2, gin2_b1, gin2_b2, mlp_b1, mlp_b2)]

    h = x
    agg = _sc_segsum(h, src_p, dst_p, zeros)
    h = _tc_mlp(h, agg, gin0_w1, b[0], gin0_w2, b[1], relu_out=True)
    agg = _sc_segsum(h, src_p, dst_p, zeros)
    h = _tc_mlp(h, agg, gin1_w1, b[2], gin1_w2, b[3], relu_out=True)
    agg = _sc_segsum(h, src_p, dst_p, zeros)
    return _tc_final(h, agg, gin2_w1, b[4], gin2_w2, b[5], batch3,
                     ego_states, mlp_w1, b[6], mlp_w2, b[7])


# speedup vs baseline: 3.0314x; 1.0148x over previous
import functools

import jax
import jax.numpy as jnp
from jax import lax
from jax.experimental import pallas as pl
from jax.experimental.pallas import tpu as pltpu
from jax.experimental.pallas import tpu_sc as plsc

N = 10000
D = 128
G = 64
E = 320000

NC = 2
NS = 16
NW = NC * NS
CE = 128
CPW = 80
E_PAD = NW * CPW * CE
RPT = 632
TRASH = N
AGG_ROWS = NS * RPT
GRP = 16

def _sc_segsum_body(h_hbm, src_hbm, dst_hbm, zeros_hbm, out_hbm,
                    sidx, didx, rows0, rows1, agg, semg0, semg1):
    c = lax.axis_index("c")
    s = lax.axis_index("s")
    w = c * NS + s
    pltpu.sync_copy(zeros_hbm, agg.at[pl.ds(s * RPT, RPT)])
    plsc.subcore_barrier()

    def group(g, carry):
        base = w * CPW + g * GRP
        pltpu.sync_copy(src_hbm.at[pl.ds(base, GRP)], sidx)
        pltpu.sync_copy(dst_hbm.at[pl.ds(base, GRP)], didx)
        pltpu.async_copy(h_hbm.at[sidx.at[0]], rows0, semg0)

        def body(jj, c2):
            j0 = 2 * jj
            j1 = j0 + 1
            jn = jnp.minimum(j0 + 2, GRP - 1)
            pltpu.make_async_copy(h_hbm.at[sidx.at[j0]], rows0, semg0).wait()
            pltpu.async_copy(h_hbm.at[sidx.at[j1]], rows1, semg1)
            pltpu.sync_copy(rows0, agg.at[didx.at[j0]], add=True)
            pltpu.make_async_copy(h_hbm.at[sidx.at[j1]], rows1, semg1).wait()
            pltpu.async_copy(h_hbm.at[sidx.at[jn]], rows0, semg0)
            pltpu.sync_copy(rows1, agg.at[didx.at[j1]], add=True)
            return c2

        lax.fori_loop(0, GRP // 2, body, 0)
        pltpu.make_async_copy(h_hbm.at[sidx.at[GRP - 1]], rows0, semg0).wait()
        return carry

    lax.fori_loop(0, CPW // GRP, group, 0)
    plsc.subcore_barrier()
    pltpu.sync_copy(agg.at[pl.ds(s * RPT, RPT)],
                    out_hbm.at[c, pl.ds(s * RPT, RPT)])


@functools.cache
def _sc_segsum_kernel():
    mesh = plsc.VectorSubcoreMesh(core_axis_name="c", subcore_axis_name="s",
                                  num_cores=NC, num_subcores=NS)
    return pl.kernel(
        _sc_segsum_body,
        out_type=jax.ShapeDtypeStruct((NC, AGG_ROWS, D), jnp.float32),
        mesh=mesh,
        scratch_types=[
            pltpu.VMEM((GRP, CE), jnp.int32),
            pltpu.VMEM((GRP, CE), jnp.int32),
            pltpu.VMEM((CE, D), jnp.float32),
            pltpu.VMEM((CE, D), jnp.float32),
            pltpu.VMEM_SHARED((AGG_ROWS, D), jnp.float32),
            pltpu.SemaphoreType.DMA,
            pltpu.SemaphoreType.DMA,
        ],
    )


def _sc_segsum(h, src_p, dst_p, zeros):
    return _sc_segsum_kernel()(h, src_p, dst_p, zeros)


BM = 1000
NB = N // BM


def _mlp_body(relu_out, h_ref, agg_ref, w1_ref, b1_ref, w2_ref, b2_ref, o_ref):
    z = h_ref[...] + agg_ref[0] + agg_ref[1]
    y = jnp.dot(z, w1_ref[...], preferred_element_type=jnp.float32) + b1_ref[...]
    y = jnp.maximum(y, 0.0)
    y = jnp.dot(y, w2_ref[...], preferred_element_type=jnp.float32) + b2_ref[...]
    if relu_out:
        y = jnp.maximum(y, 0.0)
    o_ref[...] = y


def _tc_mlp(h, agg, w1, b1, w2, b2, relu_out):
    return pl.pallas_call(
        functools.partial(_mlp_body, relu_out),
        grid=(NB,),
        in_specs=[
            pl.BlockSpec((BM, D), lambda i: (i, 0)),
            pl.BlockSpec((NC, BM, D), lambda i: (0, i, 0)),
            pl.BlockSpec((D, D), lambda i: (0, 0)),
            pl.BlockSpec((1, D), lambda i: (0, 0)),
            pl.BlockSpec((D, D), lambda i: (0, 0)),
            pl.BlockSpec((1, D), lambda i: (0, 0)),
        ],
        out_specs=pl.BlockSpec((BM, D), lambda i: (i, 0)),
        out_shape=jax.ShapeDtypeStruct((N, D), jnp.float32),
    )(h, agg, w1, b1, w2, b2)


def _final_body(h_ref, agg_ref, w1_ref, b1_ref, w2_ref, b2_ref, batch_ref,
                ego_ref, mw1_ref, mb1_ref, mw2_ref, mb2_ref, o_ref, acc, cnt):
    i = pl.program_id(0)

    @pl.when(i == 0)
    def _():
        acc[...] = jnp.zeros_like(acc)
        cnt[...] = jnp.zeros_like(cnt)

    z = h_ref[...] + agg_ref[0] + agg_ref[1]
    y = jnp.maximum(
        jnp.dot(z, w1_ref[...], preferred_element_type=jnp.float32) + b1_ref[...], 0.0)
    h3 = jnp.dot(y, w2_ref[...], preferred_element_type=jnp.float32) + b2_ref[...]

    bb = batch_ref[0]
    gi = lax.broadcasted_iota(jnp.int32, (G, BM), 0)
    pt = jnp.where(gi == bb, 1.0, 0.0)
    acc[...] += lax.dot_general(pt, h3, (((1,), (0,)), ((), ())),
                                preferred_element_type=jnp.float32)
    cnt[...] += jnp.broadcast_to(jnp.sum(pt, axis=1, keepdims=True), (G, D))

    @pl.when(i == NB - 1)
    def _():
        pooled = acc[...] / jnp.maximum(cnt[...], 1.0)
        e = jnp.maximum(
            jnp.dot(ego_ref[...], mw1_ref[...], preferred_element_type=jnp.float32)
            + mb1_ref[...], 0.0)
        e = jnp.dot(e, mw2_ref[...], preferred_element_type=jnp.float32) + mb2_ref[...]
        o_ref[...] = jnp.concatenate([pooled, e], axis=1)


def _tc_final(h, agg, w1, b1, w2, b2, batch3, ego, mw1, mb1, mw2, mb2):
    return pl.pallas_call(
        _final_body,
        grid=(NB,),
        in_specs=[
            pl.BlockSpec((BM, D), lambda i: (i, 0)),
            pl.BlockSpec((NC, BM, D), lambda i: (0, i, 0)),
            pl.BlockSpec((D, D), lambda i: (0, 0)),
            pl.BlockSpec((1, D), lambda i: (0, 0)),
            pl.BlockSpec((D, D), lambda i: (0, 0)),
            pl.BlockSpec((1, D), lambda i: (0, 0)),
            pl.BlockSpec((1, 1, BM), lambda i: (i, 0, 0)),
            pl.BlockSpec((G, D), lambda i: (0, 0)),
            pl.BlockSpec((D, D), lambda i: (0, 0)),
            pl.BlockSpec((1, D), lambda i: (0, 0)),
            pl.BlockSpec((D, D), lambda i: (0, 0)),
            pl.BlockSpec((1, D), lambda i: (0, 0)),
        ],
        out_specs=pl.BlockSpec((G, 2 * D), lambda i: (0, 0)),
        out_shape=jax.ShapeDtypeStruct((G, 2 * D), jnp.float32),
        scratch_shapes=[
            pltpu.VMEM((G, D), jnp.float32),
            pltpu.VMEM((G, D), jnp.float32),
        ],
    )(h, agg, w1, b1, w2, b2, batch3, ego, mw1, mb1, mw2, mb2)


def kernel(ego_states, x, edge_index, batch,
           gin0_w1, gin0_b1, gin0_w2, gin0_b2,
           gin1_w1, gin1_b1, gin1_w2, gin1_b2,
           gin2_w1, gin2_b1, gin2_w2, gin2_b2,
           mlp_w1, mlp_b1, mlp_w2, mlp_b2):
    pad = E_PAD - E
    src_p = jnp.concatenate(
        [edge_index[0], jnp.zeros((pad,), jnp.int32)]).reshape(NW * CPW, CE)
    dst_p = jnp.concatenate(
        [edge_index[1], jnp.full((pad,), TRASH, jnp.int32)]).reshape(NW * CPW, CE)
    zeros = jnp.zeros((RPT, D), jnp.float32)
    batch3 = batch.reshape(NB, 1, BM)

    b = [v.reshape(1, D) for v in
         (gin0_b1, gin0_b2, gin1_b1, gin1_b2, gin2_b1, gin2_b2, mlp_b1, mlp_b2)]

    h = x
    agg = _sc_segsum(h, src_p, dst_p, zeros)
    h = _tc_mlp(h, agg, gin0_w1, b[0], gin0_w2, b[1], relu_out=True)
    agg = _sc_segsum(h, src_p, dst_p, zeros)
    h = _tc_mlp(h, agg, gin1_w1, b[2], gin1_w2, b[3], relu_out=True)
    agg = _sc_segsum(h, src_p, dst_p, zeros)
    return _tc_final(h, agg, gin2_w1, b[4], gin2_w2, b[5], batch3,
                     ego_states, mlp_w1, b[6], mlp_w2, b[7])


# speedup vs baseline: 3.1678x; 1.0450x over previous
import functools

import jax
import jax.numpy as jnp
from jax import lax
from jax.experimental import pallas as pl
from jax.experimental.pallas import tpu as pltpu
from jax.experimental.pallas import tpu_sc as plsc

N = 10000
D = 128
G = 64
E = 320000

NC = 2
NS = 16
NW = NC * NS
CE = 128
CPW0 = 120
CPW1 = 40
E_PAD = NS * (CPW0 + CPW1) * CE
RPT = 632
TRASH = N
AGG_ROWS = NS * RPT
GRP = 8

def _sc_segsum_body(h_hbm, src_hbm, dst_hbm, zeros_hbm, out_hbm,
                    sidx, didx, rows0, rows1, agg, semg0, semg1):
    c = lax.axis_index("c")
    s = lax.axis_index("s")
    slab = lax.select(c == 0, s * CPW0, NS * CPW0 + s * CPW1)
    ngroups = lax.select(c == 0, CPW0 // GRP, CPW1 // GRP)
    pltpu.sync_copy(zeros_hbm, agg.at[pl.ds(s * RPT, RPT)])
    plsc.subcore_barrier()

    def group(g, carry):
        base = slab + g * GRP
        pltpu.sync_copy(src_hbm.at[pl.ds(base, GRP)], sidx)
        pltpu.sync_copy(dst_hbm.at[pl.ds(base, GRP)], didx)
        pltpu.async_copy(h_hbm.at[sidx.at[0]], rows0, semg0)

        def body(jj, c2):
            j0 = 2 * jj
            j1 = j0 + 1
            jn = jnp.minimum(j0 + 2, GRP - 1)
            pltpu.make_async_copy(h_hbm.at[sidx.at[j0]], rows0, semg0).wait()
            pltpu.async_copy(h_hbm.at[sidx.at[j1]], rows1, semg1)
            pltpu.sync_copy(rows0, agg.at[didx.at[j0]], add=True)
            pltpu.make_async_copy(h_hbm.at[sidx.at[j1]], rows1, semg1).wait()
            pltpu.async_copy(h_hbm.at[sidx.at[jn]], rows0, semg0)
            pltpu.sync_copy(rows1, agg.at[didx.at[j1]], add=True)
            return c2

        lax.fori_loop(0, GRP // 2, body, 0)
        pltpu.make_async_copy(h_hbm.at[sidx.at[GRP - 1]], rows0, semg0).wait()
        return carry

    lax.fori_loop(0, ngroups, group, 0)
    plsc.subcore_barrier()
    pltpu.sync_copy(agg.at[pl.ds(s * RPT, RPT)],
                    out_hbm.at[c, pl.ds(s * RPT, RPT)])


@functools.cache
def _sc_segsum_kernel():
    mesh = plsc.VectorSubcoreMesh(core_axis_name="c", subcore_axis_name="s",
                                  num_cores=NC, num_subcores=NS)
    return pl.kernel(
        _sc_segsum_body,
        out_type=jax.ShapeDtypeStruct((NC, AGG_ROWS, D), jnp.float32),
        mesh=mesh,
        scratch_types=[
            pltpu.VMEM((GRP, CE), jnp.int32),
            pltpu.VMEM((GRP, CE), jnp.int32),
            pltpu.VMEM((CE, D), jnp.float32),
            pltpu.VMEM((CE, D), jnp.float32),
            pltpu.VMEM_SHARED((AGG_ROWS, D), jnp.float32),
            pltpu.SemaphoreType.DMA,
            pltpu.SemaphoreType.DMA,
        ],
    )


def _sc_segsum(h, src_p, dst_p, zeros):
    return _sc_segsum_kernel()(h, src_p, dst_p, zeros)


BM = 1000
NB = N // BM


def _mlp_body(relu_out, h_ref, agg_ref, w1_ref, b1_ref, w2_ref, b2_ref, o_ref):
    z = h_ref[...] + agg_ref[0] + agg_ref[1]
    y = jnp.dot(z, w1_ref[...], preferred_element_type=jnp.float32) + b1_ref[...]
    y = jnp.maximum(y, 0.0)
    y = jnp.dot(y, w2_ref[...], preferred_element_type=jnp.float32) + b2_ref[...]
    if relu_out:
        y = jnp.maximum(y, 0.0)
    o_ref[...] = y


def _tc_mlp(h, agg, w1, b1, w2, b2, relu_out):
    return pl.pallas_call(
        functools.partial(_mlp_body, relu_out),
        grid=(NB,),
        in_specs=[
            pl.BlockSpec((BM, D), lambda i: (i, 0)),
            pl.BlockSpec((NC, BM, D), lambda i: (0, i, 0)),
            pl.BlockSpec((D, D), lambda i: (0, 0)),
            pl.BlockSpec((1, D), lambda i: (0, 0)),
            pl.BlockSpec((D, D), lambda i: (0, 0)),
            pl.BlockSpec((1, D), lambda i: (0, 0)),
        ],
        out_specs=pl.BlockSpec((BM, D), lambda i: (i, 0)),
        out_shape=jax.ShapeDtypeStruct((N, D), jnp.float32),
    )(h, agg, w1, b1, w2, b2)


def _final_body(h_ref, agg_ref, w1_ref, b1_ref, w2_ref, b2_ref, batch_ref,
                ego_ref, mw1_ref, mb1_ref, mw2_ref, mb2_ref, o_ref, acc, cnt):
    i = pl.program_id(0)

    @pl.when(i == 0)
    def _():
        acc[...] = jnp.zeros_like(acc)
        cnt[...] = jnp.zeros_like(cnt)

    z = h_ref[...] + agg_ref[0] + agg_ref[1]
    y = jnp.maximum(
        jnp.dot(z, w1_ref[...], preferred_element_type=jnp.float32) + b1_ref[...], 0.0)
    h3 = jnp.dot(y, w2_ref[...], preferred_element_type=jnp.float32) + b2_ref[...]

    bb = batch_ref[0]
    gi = lax.broadcasted_iota(jnp.int32, (G, BM), 0)
    pt = jnp.where(gi == bb, 1.0, 0.0)
    acc[...] += lax.dot_general(pt, h3, (((1,), (0,)), ((), ())),
                                preferred_element_type=jnp.float32)
    cnt[...] += jnp.broadcast_to(jnp.sum(pt, axis=1, keepdims=True), (G, D))

    @pl.when(i == NB - 1)
    def _():
        pooled = acc[...] / jnp.maximum(cnt[...], 1.0)
        e = jnp.maximum(
            jnp.dot(ego_ref[...], mw1_ref[...], preferred_element_type=jnp.float32)
            + mb1_ref[...], 0.0)
        e = jnp.dot(e, mw2_ref[...], preferred_element_type=jnp.float32) + mb2_ref[...]
        o_ref[...] = jnp.concatenate([pooled, e], axis=1)


def _tc_final(h, agg, w1, b1, w2, b2, batch3, ego, mw1, mb1, mw2, mb2):
    return pl.pallas_call(
        _final_body,
        grid=(NB,),
        in_specs=[
            pl.BlockSpec((BM, D), lambda i: (i, 0)),
            pl.BlockSpec((NC, BM, D), lambda i: (0, i, 0)),
            pl.BlockSpec((D, D), lambda i: (0, 0)),
            pl.BlockSpec((1, D), lambda i: (0, 0)),
            pl.BlockSpec((D, D), lambda i: (0, 0)),
            pl.BlockSpec((1, D), lambda i: (0, 0)),
            pl.BlockSpec((1, 1, BM), lambda i: (i, 0, 0)),
            pl.BlockSpec((G, D), lambda i: (0, 0)),
            pl.BlockSpec((D, D), lambda i: (0, 0)),
            pl.BlockSpec((1, D), lambda i: (0, 0)),
            pl.BlockSpec((D, D), lambda i: (0, 0)),
            pl.BlockSpec((1, D), lambda i: (0, 0)),
        ],
        out_specs=pl.BlockSpec((G, 2 * D), lambda i: (0, 0)),
        out_shape=jax.ShapeDtypeStruct((G, 2 * D), jnp.float32),
        scratch_shapes=[
            pltpu.VMEM((G, D), jnp.float32),
            pltpu.VMEM((G, D), jnp.float32),
        ],
    )(h, agg, w1, b1, w2, b2, batch3, ego, mw1, mb1, mw2, mb2)


def kernel(ego_states, x, edge_index, batch,
           gin0_w1, gin0_b1, gin0_w2, gin0_b2,
           gin1_w1, gin1_b1, gin1_w2, gin1_b2,
           gin2_w1, gin2_b1, gin2_w2, gin2_b2,
           mlp_w1, mlp_b1, mlp_w2, mlp_b2):
    pad = E_PAD - E
    src_p = jnp.concatenate(
        [edge_index[0], jnp.zeros((pad,), jnp.int32)]).reshape(E_PAD // CE, CE)
    dst_p = jnp.concatenate(
        [edge_index[1], jnp.full((pad,), TRASH, jnp.int32)]).reshape(E_PAD // CE, CE)
    zeros = jnp.zeros((RPT, D), jnp.float32)
    batch3 = batch.reshape(NB, 1, BM)

    b = [v.reshape(1, D) for v in
         (gin0_b1, gin0_b2, gin1_b1, gin1_b2, gin2_b1, gin2_b2, mlp_b1, mlp_b2)]

    h = x
    agg = _sc_segsum(h, src_p, dst_p, zeros)
    h = _tc_mlp(h, agg, gin0_w1, b[0], gin0_w2, b[1], relu_out=True)
    agg = _sc_segsum(h, src_p, dst_p, zeros)
    h = _tc_mlp(h, agg, gin1_w1, b[2], gin1_w2, b[3], relu_out=True)
    agg = _sc_segsum(h, src_p, dst_p, zeros)
    return _tc_final(h, agg, gin2_w1, b[4], gin2_w2, b[5], batch3,
                     ego_states, mlp_w1, b[6], mlp_w2, b[7])
